# baseline (speedup 1.0000x reference)
import jax
import jax.numpy as jnp
from jax.experimental import pallas as pl
from jax.experimental.pallas import tpu as pltpu

_EMBED = 2048
_HIDDEN = 8192
_BM = 1024
_BH = 512
_BN = 512

_INV_SQRT2 = 0.7071067811865476


def _mlp_body(x_ref, w1_ref, b1_ref, w2_ref, b2_ref, o_ref, xbs_ref):
    j = pl.program_id(1)

    @pl.when(j == 0)
    def _cast_x():
        xbs_ref[...] = x_ref[...].astype(jnp.bfloat16)

    w1b = w1_ref[...].astype(jnp.bfloat16)
    h = jax.lax.dot_general(
        xbs_ref[...], w1b,
        (((1,), (1,)), ((), ())),
        preferred_element_type=jnp.float32)
    h = h + b1_ref[...]
    h = 0.5 * h * (1.0 + jax.lax.erf(h * _INV_SQRT2))
    hb = h.astype(jnp.bfloat16)
    for n in range(_EMBED // _BN):
        nsl = pl.ds(n * _BN, _BN)
        w2b = w2_ref[nsl, :].astype(jnp.bfloat16)
        part = jax.lax.dot_general(
            hb, w2b,
            (((1,), (1,)), ((), ())),
            preferred_element_type=jnp.float32)

        @pl.when(j == 0)
        def _init():
            o_ref[:, nsl] = part + b2_ref[:, nsl]

        @pl.when(j != 0)
        def _acc():
            o_ref[:, nsl] += part


def kernel(x, proj1, proj1_bias, proj2, proj2_bias, gate_w):
    del gate_w
    L, N, E = x.shape
    M = L * N
    xf = x.reshape(M, E)
    b1 = proj1_bias.reshape(1, _HIDDEN)
    b2 = proj2_bias.reshape(1, _EMBED)

    grid = (M // _BM, _HIDDEN // _BH)
    out = pl.pallas_call(
        _mlp_body,
        grid=grid,
        in_specs=[
            pl.BlockSpec((_BM, _EMBED), lambda i, j: (i, 0)),
            pl.BlockSpec((_BH, _EMBED), lambda i, j: (j, 0)),
            pl.BlockSpec((1, _BH), lambda i, j: (0, j)),
            pl.BlockSpec((_EMBED, _BH), lambda i, j: (0, j)),
            pl.BlockSpec((1, _EMBED), lambda i, j: (0, 0)),
        ],
        out_specs=pl.BlockSpec((_BM, _EMBED), lambda i, j: (i, 0)),
        out_shape=jax.ShapeDtypeStruct((M, E), jnp.float32),
        scratch_shapes=[pltpu.VMEM((_BM, _EMBED), jnp.bfloat16)],
        compiler_params=pltpu.CompilerParams(
            dimension_semantics=("parallel", "arbitrary"),
            vmem_limit_bytes=66_000_000,
        ),
    )(xf, proj1, b1, proj2, b2)
    return out.reshape(L, N, E)


# speedup vs baseline: 1.0868x; 1.0868x over previous
import jax
import jax.numpy as jnp
from jax.experimental import pallas as pl
from jax.experimental.pallas import tpu as pltpu

_EMBED = 2048
_HIDDEN = 8192
_BM = 1024
_BH = 512

_INV_SQRT2 = 0.7071067811865476


def _mlp_body(x_ref, w1_ref, b1_ref, w2_ref, b2_ref, o_ref):
    j = pl.program_id(1)
    w1b = w1_ref[...].astype(jnp.bfloat16)
    h = jax.lax.dot_general(
        x_ref[...], w1b,
        (((1,), (1,)), ((), ())),
        preferred_element_type=jnp.float32)
    h = h + b1_ref[...]
    h = 0.5 * h * (1.0 + jax.lax.erf(h * _INV_SQRT2))
    w2b = w2_ref[...].astype(jnp.bfloat16)
    contrib = jax.lax.dot_general(
        h.astype(jnp.bfloat16), w2b,
        (((1,), (1,)), ((), ())),
        preferred_element_type=jnp.float32)

    @pl.when(j == 0)
    def _init():
        o_ref[...] = contrib + b2_ref[...]

    @pl.when(j != 0)
    def _acc():
        o_ref[...] += contrib


def kernel(x, proj1, proj1_bias, proj2, proj2_bias, gate_w):
    del gate_w
    L, N, E = x.shape
    M = L * N
    xb = x.reshape(M, E).astype(jnp.bfloat16)
    b1 = proj1_bias.reshape(1, _HIDDEN)
    b2 = proj2_bias.reshape(1, _EMBED)

    grid = (M // _BM, _HIDDEN // _BH)
    out = pl.pallas_call(
        _mlp_body,
        grid=grid,
        in_specs=[
            pl.BlockSpec((_BM, _EMBED), lambda i, j: (i, 0)),
            pl.BlockSpec((_BH, _EMBED), lambda i, j: (j, 0)),
            pl.BlockSpec((1, _BH), lambda i, j: (0, j)),
            pl.BlockSpec((_EMBED, _BH), lambda i, j: (0, j)),
            pl.BlockSpec((1, _EMBED), lambda i, j: (0, 0)),
        ],
        out_specs=pl.BlockSpec((_BM, _EMBED), lambda i, j: (i, 0)),
        out_shape=jax.ShapeDtypeStruct((M, E), jnp.float32),
        compiler_params=pltpu.CompilerParams(
            dimension_semantics=("parallel", "arbitrary"),
        ),
    )(xb, proj1, b1, proj2, b2)
    return out.reshape(L, N, E)
